# Initial kernel scaffold; baseline (speedup 1.0000x reference)
import functools

import jax
import jax.numpy as jnp
from jax import lax
from jax.experimental import pallas as pl
from jax.experimental.pallas import tpu as pltpu
from jax.experimental.pallas import tpu_sc as plsc

_COARSE = 100
_EVENT = 1000
_EVENT_PAD = 1024
_BETA = 1.0

_NC = 2
_NS = 16
_L = 16
_NW = _NC * _NS


def _table_body(ftc_ref, cf_ref, w_ref, out_ref):
    ftc = ftc_ref[...]
    cf = cf_ref[...]
    w = w_ref[0:1, :]
    kidx = lax.broadcasted_iota(jnp.int32, (_EVENT_PAD, 1), 0)
    valid = kidx < _EVENT
    cid = lax.broadcasted_iota(jnp.int32, (_EVENT_PAD, 128), 1)
    onehot = (ftc == cid) & valid
    onehotf = onehot.astype(jnp.float32)
    neg = jnp.float32(-1e30)
    gmax_c = jnp.max(jnp.where(onehot, cf, neg), axis=0, keepdims=True)
    gmax_k = jnp.sum(onehotf * gmax_c, axis=1, keepdims=True)
    ex = jnp.where(valid, jnp.exp(cf - gmax_k), 0.0)
    mass_c = jnp.sum(onehotf * ex, axis=0, keepdims=True)
    mass_k = jnp.sum(onehotf * mass_c, axis=1, keepdims=True)
    w_k = jnp.sum(onehotf * w, axis=1, keepdims=True)
    coarse_int = jax.nn.softplus(_BETA * w_k) / _BETA
    safe_mass = jnp.where(mass_k > 0.0, mass_k, 1.0)
    out_ref[...] = coarse_int * ex / safe_mass


_table_call = pl.pallas_call(
    _table_body,
    out_shape=jax.ShapeDtypeStruct((_EVENT_PAD, 1), jnp.float32),
)


def _make_gather_call(total):
    b_per_w = total // _NW
    mesh = plsc.VectorSubcoreMesh(core_axis_name="c", subcore_axis_name="s")

    @functools.partial(
        pl.kernel,
        mesh=mesh,
        out_type=jax.ShapeDtypeStruct((total,), jnp.float32),
        scratch_types=[
            pltpu.VMEM((_EVENT_PAD,), jnp.float32),
            pltpu.VMEM((b_per_w,), jnp.int32),
            pltpu.VMEM((b_per_w,), jnp.float32),
        ],
    )
    def gather_kernel(table_hbm, idx_hbm, out_hbm, table_v, idx_v, out_v):
        wid = lax.axis_index("s") * _NC + lax.axis_index("c")
        base = wid * b_per_w
        pltpu.sync_copy(table_hbm, table_v)
        pltpu.sync_copy(idx_hbm.at[pl.ds(base, b_per_w)], idx_v)

        def body(i, carry):
            off = i * _L
            idxv = idx_v[pl.ds(off, _L)]
            out_v[pl.ds(off, _L)] = plsc.load_gather(table_v, [idxv])
            return carry

        lax.fori_loop(0, b_per_w // _L, body, 0, unroll=8)
        pltpu.sync_copy(out_v, out_hbm.at[pl.ds(base, b_per_w)])

    return gather_kernel


def kernel(event_tensor, out_emb_weight, cf_logits, fine_to_coarse):
    ftc_pad = (
        jnp.zeros((_EVENT_PAD,), jnp.int32)
        .at[:_EVENT]
        .set(fine_to_coarse.astype(jnp.int32))
        .reshape(_EVENT_PAD, 1)
    )
    cf_pad = (
        jnp.zeros((_EVENT_PAD,), jnp.float32)
        .at[:_EVENT]
        .set(cf_logits.astype(jnp.float32))
        .reshape(_EVENT_PAD, 1)
    )
    w_pad = jnp.zeros((8, 128), jnp.float32).at[0, :_COARSE].set(
        out_emb_weight[:, 0].astype(jnp.float32)
    )
    table = _table_call(ftc_pad, cf_pad, w_pad).reshape(_EVENT_PAD)

    idx = event_tensor.reshape(-1).astype(jnp.int32)
    out = _make_gather_call(idx.shape[0])(table, idx)
    return out.reshape(event_tensor.shape)


# speedup vs baseline: 97.3411x; 97.3411x over previous
import functools

import jax
import jax.numpy as jnp
from jax import lax
from jax.experimental import pallas as pl
from jax.experimental.pallas import tpu as pltpu
from jax.experimental.pallas import tpu_sc as plsc

_COARSE = 100
_EVENT = 1000
_EVENT_PAD = 1024
_BETA = 1.0

_NC = 2
_NS = 16
_L = 16
_NW = _NC * _NS


def _table_body(ftc_ref, cf_ref, w_ref, out_ref):
    ftc = ftc_ref[...]
    cf = cf_ref[...]
    w = w_ref[0:1, :]
    kidx = lax.broadcasted_iota(jnp.int32, (_EVENT_PAD, 1), 0)
    valid = kidx < _EVENT
    cid = lax.broadcasted_iota(jnp.int32, (_EVENT_PAD, 128), 1)
    onehot = (ftc == cid) & valid
    onehotf = onehot.astype(jnp.float32)
    neg = jnp.float32(-1e30)
    gmax_c = jnp.max(jnp.where(onehot, cf, neg), axis=0, keepdims=True)
    gmax_k = jnp.sum(onehotf * gmax_c, axis=1, keepdims=True)
    ex = jnp.where(valid, jnp.exp(cf - gmax_k), 0.0)
    mass_c = jnp.sum(onehotf * ex, axis=0, keepdims=True)
    mass_k = jnp.sum(onehotf * mass_c, axis=1, keepdims=True)
    w_k = jnp.sum(onehotf * w, axis=1, keepdims=True)
    coarse_int = jax.nn.softplus(_BETA * w_k) / _BETA
    safe_mass = jnp.where(mass_k > 0.0, mass_k, 1.0)
    out_ref[...] = coarse_int * ex / safe_mass


_table_call = pl.pallas_call(
    _table_body,
    out_shape=jax.ShapeDtypeStruct((_EVENT_PAD, 1), jnp.float32),
)


def _make_gather_call(total):
    b_per_w = total // _NW
    mesh = plsc.VectorSubcoreMesh(core_axis_name="c", subcore_axis_name="s")

    @functools.partial(
        pl.kernel,
        mesh=mesh,
        out_type=jax.ShapeDtypeStruct((total,), jnp.float32),
        scratch_types=[
            pltpu.VMEM((_EVENT_PAD,), jnp.float32),
            pltpu.VMEM((b_per_w,), jnp.int32),
            pltpu.VMEM((b_per_w,), jnp.float32),
        ],
        compiler_params=pltpu.CompilerParams(needs_layout_passes=False),
    )
    def gather_kernel(table_hbm, idx_hbm, out_hbm, table_v, idx_v, out_v):
        wid = lax.axis_index("s") * _NC + lax.axis_index("c")
        base = wid * b_per_w
        pltpu.sync_copy(table_hbm, table_v)
        pltpu.sync_copy(idx_hbm.at[pl.ds(base, b_per_w)], idx_v)

        def body(i, carry):
            off = i * _L
            idxv = idx_v[pl.ds(off, _L)]
            out_v[pl.ds(off, _L)] = plsc.load_gather(table_v, [idxv])
            return carry

        lax.fori_loop(0, b_per_w // _L, body, 0, unroll=8)
        pltpu.sync_copy(out_v, out_hbm.at[pl.ds(base, b_per_w)])

    return gather_kernel


def kernel(event_tensor, out_emb_weight, cf_logits, fine_to_coarse):
    ftc_pad = (
        jnp.zeros((_EVENT_PAD,), jnp.int32)
        .at[:_EVENT]
        .set(fine_to_coarse.astype(jnp.int32))
        .reshape(_EVENT_PAD, 1)
    )
    cf_pad = (
        jnp.zeros((_EVENT_PAD,), jnp.float32)
        .at[:_EVENT]
        .set(cf_logits.astype(jnp.float32))
        .reshape(_EVENT_PAD, 1)
    )
    w_pad = jnp.zeros((8, 128), jnp.float32).at[0, :_COARSE].set(
        out_emb_weight[:, 0].astype(jnp.float32)
    )
    table = _table_call(ftc_pad, cf_pad, w_pad).reshape(_EVENT_PAD)

    idx = event_tensor.reshape(-1).astype(jnp.int32)
    out = _make_gather_call(idx.shape[0])(table, idx)
    return out.reshape(event_tensor.shape)
